# baseline (speedup 1.0000x reference)
import functools
import jax
import jax.numpy as jnp
from jax import lax
from jax.experimental import pallas as pl
from jax.experimental.pallas import tpu as pltpu
from jax.experimental.pallas import tpu_sc as plsc

_NS = 10000
_NA = 10000
_E = 320000
_H = 128
_M = 64

_BN = 2000
_BE = 12800

_NC = 2
_NT = 16
_NW = _NC * _NT
_NSP = 10240
_ROWS_PER_TILE = _NSP // _NT

_mesh = lambda: plsc.VectorSubcoreMesh(core_axis_name="c", subcore_axis_name="s")


def _node_mlp_body(u_ref, h_ref,
                   uuW1, uuW2, uuW3,
                   hhW1, hhb1, hhW2, hhb2, hhW3, hhb3,
                   s_out, msg_out):
    f32 = jnp.float32
    s = jnp.tanh(jnp.dot(u_ref[...], uuW1[...], preferred_element_type=f32))
    s = jnp.tanh(jnp.dot(s, uuW2[...], preferred_element_type=f32))
    s_out[...] = jnp.dot(s, uuW3[...], preferred_element_type=f32)
    m = jnp.tanh(jnp.dot(h_ref[...], hhW1[...], preferred_element_type=f32) + hhb1[...])
    m = jnp.tanh(jnp.dot(m, hhW2[...], preferred_element_type=f32) + hhb2[...])
    msg_out[...] = jnp.dot(m, hhW3[...], preferred_element_type=f32) + hhb3[...]


def _node_mlps(u, h, p):
    grid = (_NS // _BN,)
    bspec = pl.BlockSpec((_BN, _H), lambda i: (i, 0))
    wspec = lambda shape: pl.BlockSpec(shape, lambda i: tuple(0 for _ in shape))
    out_shape = [jax.ShapeDtypeStruct((_NS, _H), jnp.float32)] * 2
    return pl.pallas_call(
        _node_mlp_body,
        grid=grid,
        in_specs=[bspec, bspec,
                  wspec((_H, _M)), wspec((_M, _M)), wspec((_M, _H)),
                  wspec((_H, _M)), wspec((_M,)), wspec((_M, _M)),
                  wspec((_M,)), wspec((_M, _H)), wspec((_H,))],
        out_specs=[bspec, bspec],
        out_shape=out_shape,
    )(u, h, p['uu_W1'], p['uu_W2'], p['uu_W3'],
      p['hh_W1'], p['hh_b1'], p['hh_W2'], p['hh_b2'], p['hh_W3'], p['hh_b3'])


def _pack_rows(z, n):
    z3 = jnp.reshape(z, (n // 2, 2, _H))
    ue = lax.bitcast_convert_type(
        lax.convert_element_type(z3[:, 0, :], jnp.bfloat16), jnp.uint16)
    uo = lax.bitcast_convert_type(
        lax.convert_element_type(z3[:, 1, :], jnp.bfloat16), jnp.uint16)
    u = ue.astype(jnp.uint32) | (uo.astype(jnp.uint32) << 16)
    return lax.bitcast_convert_type(u, jnp.int32)


def _edge_mlp_body(act, in5_ref, W1, b1, W2, b2, W3, b3, *outs):
    f32 = jnp.float32
    pre = lax.dot_general(in5_ref[...], W1[...],
                          dimension_numbers=(((0,), (0,)), ((), ())),
                          preferred_element_type=f32)
    t = jnp.tanh(pre + b1[...])
    t = jnp.tanh(jnp.dot(t, W2[...], preferred_element_type=f32) + b2[...])
    z = jnp.dot(t, W3[...], preferred_element_type=f32) + b3[...]
    if act == 'sigmoid':
        z = jax.nn.sigmoid(z)
        outs[0][...] = _pack_rows(z, _BE)
    else:
        z = jnp.exp(z)
        outs[0][...] = _pack_rows(z, _BE)
        outs[1][...] = z


def _edge_mlp(in5t, W1, b1, W2, b2, W3, b3, act):
    grid = (_E // _BE,)
    wspec = lambda shape: pl.BlockSpec(shape, lambda i: tuple(0 for _ in shape))
    pk_spec = pl.BlockSpec((_BE // 2, _H), lambda i: (i, 0))
    pk_shape = jax.ShapeDtypeStruct((_E // 2, _H), jnp.int32)
    if act == 'sigmoid':
        out_specs, out_shape = pk_spec, pk_shape
    else:
        out_specs = [pk_spec, pl.BlockSpec((_BE, _H), lambda i: (i, 0))]
        out_shape = [pk_shape, jax.ShapeDtypeStruct((_E, _H), jnp.float32)]
    return pl.pallas_call(
        functools.partial(_edge_mlp_body, act),
        grid=grid,
        in_specs=[pl.BlockSpec((5, _BE), lambda i: (0, i)),
                  wspec((5, _M)), wspec((_M,)), wspec((_M, _M)),
                  wspec((_M,)), wspec((_M, _H)), wspec((_H,))],
        out_specs=out_specs,
        out_shape=out_shape,
    )(in5t, W1, b1, W2, b2, W3, b3)


def _update_body(sp_ref, h_ref, su_ref, num_ref, den_ref,
                 W1a, W1b, W1c, W1d, b1, W2, b2, W3, b3, out_ref):
    f32 = jnp.float32
    su = su_ref[0] + su_ref[1]
    sum_h = num_ref[...] / (den_ref[...] + 1e-16)
    pre = (jnp.dot(sp_ref[...], W1a[...], preferred_element_type=f32)
           + jnp.dot(h_ref[...], W1b[...], preferred_element_type=f32)
           + jnp.dot(su, W1c[...], preferred_element_type=f32)
           + jnp.dot(sum_h, W1d[...], preferred_element_type=f32)
           + b1[...])
    t = jnp.tanh(pre)
    t = jnp.tanh(jnp.dot(t, W2[...], preferred_element_type=f32) + b2[...])
    out_ref[...] = jnp.dot(t, W3[...], preferred_element_type=f32) + b3[...]


def _update_mlp(sp, h, su_part, num, den, p):
    W1 = p['up_W1']
    grid = (_NS // _BN,)
    bspec = pl.BlockSpec((_BN, _H), lambda i: (i, 0))
    wspec = lambda shape: pl.BlockSpec(shape, lambda i: tuple(0 for _ in shape))
    return pl.pallas_call(
        _update_body,
        grid=grid,
        in_specs=[pl.BlockSpec((_BN, 2), lambda i: (i, 0)),
                  bspec,
                  pl.BlockSpec((_NC, _BN, _H), lambda i: (0, i, 0)),
                  bspec, bspec,
                  wspec((2, _M)), wspec((_H, _M)), wspec((_H, _M)), wspec((_H, _M)),
                  wspec((_M,)), wspec((_M, _M)), wspec((_M,)), wspec((_M, _H)), wspec((_H,))],
        out_specs=bspec,
        out_shape=jax.ShapeDtypeStruct((_NS, _H), jnp.float32),
    )(sp, h, su_part, num, den,
      W1[0:2], W1[2:130], W1[130:258], W1[258:386],
      p['up_b1'], p['up_W2'], p['up_b2'], p['up_W3'], p['up_b3'])


_C5 = 2000
_G5 = _C5 // 16
_N5 = _E // _NW // _C5


def _in5_body(stage_ap, ap_hbm, sp_hbm, src_hbm, dst_hbm, dis_hbm, out_hbm,
              ap_v, sp_v, src_v, dst_v, dis_v, out_v):
    cid = lax.axis_index("c")
    sid = lax.axis_index("s")
    wid = sid * _NC + cid
    base = wid * (_E // _NW)

    if stage_ap:
        pltpu.sync_copy(ap_hbm, ap_v)
        src_tab = ap_v
    else:
        src_tab = sp_v
    pltpu.sync_copy(sp_hbm, sp_v)
    dst_tab = sp_v

    iota = lax.iota(jnp.int32, 16)

    def chunk_body(k, _):
        off = base + k * _C5
        pltpu.sync_copy(src_hbm.at[pl.ds(off, _C5)], src_v)
        pltpu.sync_copy(dst_hbm.at[pl.ds(off, _C5)], dst_v)
        pltpu.sync_copy(dis_hbm.at[pl.ds(off, _C5)], dis_v)

        def group_body(g, _):
            isrc = src_v[pl.ds(g * 16, 16)]
            idst = dst_v[pl.ds(g * 16, 16)]
            xs = plsc.load_gather(src_tab, [isrc * 2])
            ys = plsc.load_gather(src_tab, [isrc * 2 + 1])
            xd = plsc.load_gather(dst_tab, [idst * 2])
            yd = plsc.load_gather(dst_tab, [idst * 2 + 1])
            d = dis_v[pl.ds(g * 16, 16)]
            lanes = g * 16 + iota
            plsc.store_scatter(out_v, [lanes], xs)
            plsc.store_scatter(out_v, [_C5 + lanes], ys)
            plsc.store_scatter(out_v, [2 * _C5 + lanes], xd)
            plsc.store_scatter(out_v, [3 * _C5 + lanes], yd)
            plsc.store_scatter(out_v, [4 * _C5 + lanes], d)
            return 0

        lax.fori_loop(0, _G5, group_body, 0)
        for c in range(5):
            pltpu.sync_copy(out_v.at[pl.ds(c * _C5, _C5)],
                            out_hbm.at[pl.ds(c * _E + off, _C5)])
        return 0

    lax.fori_loop(0, _N5, chunk_body, 0)


def _sc_in5(stage_ap, ap_flat, sp_flat, src, dst, dis):
    f = pl.kernel(
        functools.partial(_in5_body, stage_ap),
        out_type=jax.ShapeDtypeStruct((5 * _E,), jnp.float32),
        mesh=_mesh(),
        scratch_types=[
            pltpu.VMEM((2 * _NA,), jnp.float32),
            pltpu.VMEM((2 * _NS,), jnp.float32),
            pltpu.VMEM((_C5,), jnp.int32),
            pltpu.VMEM((_C5,), jnp.int32),
            pltpu.VMEM((_C5,), jnp.float32),
            pltpu.VMEM((5 * _C5,), jnp.float32),
        ],
        compiler_params=pltpu.CompilerParams(needs_layout_passes=False),
    )
    return f(ap_flat, sp_flat, src, dst, dis)


_CA = 80
_BOUNCE = 40


def _zero_acc_stripe(zeros_hbm, acc, sid):
    pltpu.sync_copy(zeros_hbm,
                    acc.at[pl.ds(sid * _ROWS_PER_TILE, _ROWS_PER_TILE)])


def _dump_acc_stripe(acc, sid, dst_hbm_slice_fn):
    off = sid * _ROWS_PER_TILE
    pltpu.sync_copy(acc.at[pl.ds(off, _ROWS_PER_TILE)], dst_hbm_slice_fn(off))


def _pipe_gather_mul_scatter(nch, base, lin_hbm, src_hbm, dst_hbm, tab_hbm, acc,
                             isrc, idst, sidx, rows, lin, si, sd, ss):
    def off(k):
        return base + k * _CA

    def loff(k):
        return pl.multiple_of((base + k * _CA) // 2, 8)

    pltpu.sync_copy(src_hbm.at[pl.ds(off(0), _CA)], isrc[0])
    pltpu.sync_copy(dst_hbm.at[pl.ds(off(0), _CA)], idst[0])
    pltpu.async_copy(tab_hbm.at[isrc[0]], rows[0], sd[0])
    pltpu.async_copy(lin_hbm.at[pl.ds(loff(0), _CA // 2)], lin[0], sd[0])
    pltpu.async_copy(src_hbm.at[pl.ds(off(1), _CA)], isrc[1], si[1])
    pltpu.async_copy(dst_hbm.at[pl.ds(off(1), _CA)], idst[1], si[1])

    def block(k, b):
        pltpu.make_async_copy(tab_hbm.at[isrc[b]], rows[b], sd[b]).wait()
        pltpu.make_async_copy(lin_hbm.at[pl.ds(loff(k), _CA // 2)],
                              lin[b], sd[b]).wait()

        nb = 1 - b

        @pl.when(k + 1 < nch)
        def _():
            @pl.when(k >= 1)
            def _():
                pltpu.make_async_copy(rows[nb], acc.at[sidx[nb]], ss[nb]).wait()
            pltpu.make_async_copy(src_hbm.at[pl.ds(off(k + 1), _CA)],
                                  isrc[nb], si[nb]).wait()
            pltpu.make_async_copy(dst_hbm.at[pl.ds(off(k + 1), _CA)],
                                  idst[nb], si[nb]).wait()
            pltpu.async_copy(tab_hbm.at[isrc[nb]], rows[nb], sd[nb])
            pltpu.async_copy(lin_hbm.at[pl.ds(loff(k + 1), _CA // 2)],
                             lin[nb], sd[nb])

        hi_mask = jnp.full((16,), -65536, jnp.int32)

        @plsc.parallel_loop(0, _CA // 2, unroll=4)
        def _(j):
            for c in range(_H // 16):
                s = pl.ds(c * 16, 16)
                li = lin[b][j, s]
                fe = plsc.bitcast(li << 16, jnp.float32)
                fo = plsc.bitcast(li & hi_mask, jnp.float32)
                rows[b][2 * j, s] = rows[b][2 * j, s] * fe
                rows[b][2 * j + 1, s] = rows[b][2 * j + 1, s] * fo

        for c in range(_CA // 16):
            s = pl.ds(c * 16, 16)
            sidx[b][s] = idst[b][s]
        pltpu.async_copy(rows[b], acc.at[sidx[b]], ss[b], add=True)

        @pl.when(k + 2 < nch)
        def _():
            pltpu.async_copy(src_hbm.at[pl.ds(off(k + 2), _CA)], isrc[b], si[b])
            pltpu.async_copy(dst_hbm.at[pl.ds(off(k + 2), _CA)], idst[b], si[b])

    def pair(j, _):
        k0 = 2 * j
        block(k0, 0)

        @pl.when(k0 + 1 < nch)
        def _():
            block(k0 + 1, 1)

        return 0

    lax.fori_loop(0, (nch + 1) // 2, pair, 0)
    pltpu.make_async_copy(rows[0], acc.at[sidx[0]], ss[0]).wait()
    pltpu.make_async_copy(rows[1], acc.at[sidx[1]], ss[1]).wait()


_AGG_SCRATCH = [
    pltpu.VMEM((_CA,), jnp.int32), pltpu.VMEM((_CA,), jnp.int32),
    pltpu.VMEM((_CA,), jnp.int32), pltpu.VMEM((_CA,), jnp.int32),
    pltpu.VMEM((_CA,), jnp.int32), pltpu.VMEM((_CA,), jnp.int32),
    pltpu.VMEM((_CA, _H), jnp.float32), pltpu.VMEM((_CA, _H), jnp.float32),
    pltpu.VMEM((_CA, _H), jnp.float32),
    pltpu.VMEM((_CA // 2, _H), jnp.int32), pltpu.VMEM((_CA // 2, _H), jnp.int32),
    pltpu.VMEM_SHARED((_NSP, _H), jnp.float32),
    pltpu.SemaphoreType.DMA, pltpu.SemaphoreType.DMA,
    pltpu.SemaphoreType.DMA, pltpu.SemaphoreType.DMA,
    pltpu.SemaphoreType.DMA, pltpu.SemaphoreType.DMA,
]


_EA0 = 43 * _NT * _CA


def _agg_body(gate_hbm, asrc_hbm, adst_hbm, sall_hbm,
              ex_hbm, exf_hbm, ssrc_hbm, sdst_hbm, msg_hbm,
              zeros_hbm, su_hbm, num_hbm, den_hbm,
              isrc0, isrc1, idst0, idst1, sidx0, sidx1,
              rows0, rows1, rows2, lin0, lin1, acc,
              si0, si1, sd0, sd1, ss0, ss1):
    cid = lax.axis_index("c")
    sid = lax.axis_index("s")
    bufs = ((isrc0, isrc1), (idst0, idst1), (sidx0, sidx1),
            (rows0, rows1), (lin0, lin1),
            (si0, si1), (sd0, sd1), (ss0, ss1))

    _zero_acc_stripe(zeros_hbm, acc, sid)
    plsc.subcore_barrier()

    @pl.when(cid == 0)
    def _():
        _pipe_gather_mul_scatter(_EA0 // _NT // _CA, sid * (_EA0 // _NT),
                                 gate_hbm, asrc_hbm, adst_hbm, sall_hbm, acc,
                                 *bufs)

    @pl.when(cid == 1)
    def _():
        _pipe_gather_mul_scatter((_E - _EA0) // _NT // _CA,
                                 _EA0 + sid * ((_E - _EA0) // _NT),
                                 gate_hbm, asrc_hbm, adst_hbm, sall_hbm, acc,
                                 *bufs)

    plsc.subcore_barrier()
    _dump_acc_stripe(acc, sid,
                     lambda off: su_hbm.at[cid, pl.ds(off, _ROWS_PER_TILE)])

    _zero_acc_stripe(zeros_hbm, acc, sid)
    plsc.subcore_barrier()

    base = sid * (_E // _NT)

    @pl.when(cid == 0)
    def _():
        _pipe_gather_mul_scatter(_E // _NT // _CA, base,
                                 ex_hbm, ssrc_hbm, sdst_hbm, msg_hbm, acc,
                                 *bufs)

    @pl.when(cid == 1)
    def _():
        nch = _E // _NT // _CA
        idx3 = (idst0, idst1, isrc0)
        lin3 = (rows0, rows1, rows2)
        ld3 = (sd0, sd1, si0)
        sc3 = (ss0, ss1, si1)

        def off(k):
            return base + k * _CA

        pltpu.sync_copy(sdst_hbm.at[pl.ds(off(0), _CA)], idx3[0])
        pltpu.sync_copy(exf_hbm.at[pl.ds(off(0), _CA)], lin3[0].at[pl.ds(0, _CA)])
        pltpu.async_copy(sdst_hbm.at[pl.ds(off(1), _CA)], idx3[1], ld3[1])
        pltpu.async_copy(exf_hbm.at[pl.ds(off(1), _CA)], lin3[1].at[pl.ds(0, _CA)], ld3[1])

        def den_block(k, r):
            @pl.when(k >= 1)
            def _():
                pltpu.make_async_copy(sdst_hbm.at[pl.ds(off(k), _CA)],
                                      idx3[r], ld3[r]).wait()
                pltpu.make_async_copy(exf_hbm.at[pl.ds(off(k), _CA)],
                                      lin3[r].at[pl.ds(0, _CA)], ld3[r]).wait()
            pltpu.async_copy(lin3[r].at[pl.ds(0, _CA)], acc.at[idx3[r]],
                             sc3[r], add=True)
            r2 = (r + 2) % 3

            @pl.when(k + 2 < nch)
            def _():
                @pl.when(k >= 1)
                def _():
                    pltpu.make_async_copy(lin3[r2].at[pl.ds(0, _CA)],
                                          acc.at[idx3[r2]], sc3[r2]).wait()
                pltpu.async_copy(sdst_hbm.at[pl.ds(off(k + 2), _CA)],
                                 idx3[r2], ld3[r2])
                pltpu.async_copy(exf_hbm.at[pl.ds(off(k + 2), _CA)],
                                 lin3[r2].at[pl.ds(0, _CA)], ld3[r2])

        def triple(j, _):
            k0 = 3 * j
            for r in range(3):
                @pl.when(k0 + r < nch)
                def _():
                    den_block(k0 + r, r)
            return 0

        lax.fori_loop(0, (nch + 2) // 3, triple, 0)
        for r in range(3):
            pltpu.make_async_copy(lin3[r].at[pl.ds(0, _CA)],
                                  acc.at[idx3[r]], sc3[r]).wait()

    plsc.subcore_barrier()

    @pl.when(cid == 0)
    def _():
        _dump_acc_stripe(acc, sid,
                         lambda off: num_hbm.at[pl.ds(off, _ROWS_PER_TILE)])

    @pl.when(cid == 1)
    def _():
        _dump_acc_stripe(acc, sid,
                         lambda off: den_hbm.at[pl.ds(off, _ROWS_PER_TILE)])


def _sc_agg(gate, a_src, a_dst, s_all, ex_pk, ex_f, s_src, s_dst, msg_all, zeros_stripe):
    f = pl.kernel(
        _agg_body,
        out_type=[jax.ShapeDtypeStruct((_NC, _NSP, _H), jnp.float32),
                  jax.ShapeDtypeStruct((_NSP, _H), jnp.float32),
                  jax.ShapeDtypeStruct((_NSP, _H), jnp.float32)],
        mesh=_mesh(),
        scratch_types=list(_AGG_SCRATCH),
        compiler_params=pltpu.CompilerParams(needs_layout_passes=False),
    )
    return f(gate, a_src, a_dst, s_all, ex_pk, ex_f, s_src, s_dst, msg_all, zeros_stripe)


def kernel(h, u, state_pos, action_pos, a2s_src, a2s_dst, a2s_dis,
           s2s_src, s2s_dst, s2s_dis, params):
    p = params
    ap_flat = jnp.reshape(action_pos, (-1,))
    sp_flat = jnp.reshape(state_pos, (-1,))
    a_dis = jnp.reshape(a2s_dis, (-1,))
    s_dis = jnp.reshape(s2s_dis, (-1,))

    in5a_flat = _sc_in5(True, ap_flat, sp_flat, a2s_src, a2s_dst, a_dis)
    in5a = jnp.reshape(in5a_flat, (5, _E))
    gate = _edge_mlp(in5a, p['ud_W1'], p['ud_b1'], p['ud_W2'], p['ud_b2'],
                     p['ud_W3'], p['ud_b3'], 'sigmoid')

    in5s_flat = _sc_in5(False, ap_flat, sp_flat, s2s_src, s2s_dst, s_dis)
    in5s = jnp.reshape(in5s_flat, (5, _E))
    ex_pk, ex_f = _edge_mlp(in5s, p['hd_W1'], p['hd_b1'], p['hd_W2'], p['hd_b2'],
                            p['hd_W3'], p['hd_b3'], 'exp')

    s_all, msg_all = _node_mlps(u, h, p)

    zeros_stripe = jnp.zeros((_ROWS_PER_TILE, _H), jnp.float32)
    su_part, num, den = _sc_agg(gate, a2s_src, a2s_dst, s_all,
                                ex_pk, ex_f, s2s_src, s2s_dst, msg_all, zeros_stripe)

    return _update_mlp(state_pos, h, su_part, num, den, p)


# speedup vs baseline: 1.3663x; 1.3663x over previous
import functools
import jax
import jax.numpy as jnp
from jax import lax
from jax.experimental import pallas as pl
from jax.experimental.pallas import tpu as pltpu
from jax.experimental.pallas import tpu_sc as plsc

_NS = 10000
_NA = 10000
_E = 320000
_H = 128
_M = 64

_BN = 2000
_BE = 12800

_NC = 2
_NT = 16
_NW = _NC * _NT
_NSP = 10240
_ROWS_PER_TILE = _NSP // _NT

_mesh = lambda: plsc.VectorSubcoreMesh(core_axis_name="c", subcore_axis_name="s")


def _node_mlp_body(u_ref, h_ref,
                   uuW1, uuW2, uuW3,
                   hhW1, hhb1, hhW2, hhb2, hhW3, hhb3,
                   s_out, msg_out):
    f32 = jnp.float32
    s = jnp.tanh(jnp.dot(u_ref[...], uuW1[...], preferred_element_type=f32))
    s = jnp.tanh(jnp.dot(s, uuW2[...], preferred_element_type=f32))
    s_out[...] = jnp.dot(s, uuW3[...], preferred_element_type=f32)
    m = jnp.tanh(jnp.dot(h_ref[...], hhW1[...], preferred_element_type=f32) + hhb1[...])
    m = jnp.tanh(jnp.dot(m, hhW2[...], preferred_element_type=f32) + hhb2[...])
    msg_out[...] = jnp.dot(m, hhW3[...], preferred_element_type=f32) + hhb3[...]


def _node_mlps(u, h, p):
    grid = (_NS // _BN,)
    bspec = pl.BlockSpec((_BN, _H), lambda i: (i, 0))
    wspec = lambda shape: pl.BlockSpec(shape, lambda i: tuple(0 for _ in shape))
    out_shape = [jax.ShapeDtypeStruct((_NS, _H), jnp.float32)] * 2
    return pl.pallas_call(
        _node_mlp_body,
        grid=grid,
        in_specs=[bspec, bspec,
                  wspec((_H, _M)), wspec((_M, _M)), wspec((_M, _H)),
                  wspec((_H, _M)), wspec((_M,)), wspec((_M, _M)),
                  wspec((_M,)), wspec((_M, _H)), wspec((_H,))],
        out_specs=[bspec, bspec],
        out_shape=out_shape,
    )(u, h, p['uu_W1'], p['uu_W2'], p['uu_W3'],
      p['hh_W1'], p['hh_b1'], p['hh_W2'], p['hh_b2'], p['hh_W3'], p['hh_b3'])


def _edge_mlp_body(act, in5_ref, W1, b1, W2, b2, W3, b3, out_ref):
    f32 = jnp.float32
    pre = lax.dot_general(in5_ref[...], W1[...],
                          dimension_numbers=(((0,), (0,)), ((), ())),
                          preferred_element_type=f32)
    t = jnp.tanh(pre + b1[...])
    t = jnp.tanh(jnp.dot(t, W2[...], preferred_element_type=f32) + b2[...])
    z = jnp.dot(t, W3[...], preferred_element_type=f32) + b3[...]
    if act == 'sigmoid':
        out_ref[...] = jax.nn.sigmoid(z)
    else:
        out_ref[...] = jnp.exp(z)


def _edge_mlp(in5t, W1, b1, W2, b2, W3, b3, act):
    grid = (_E // _BE,)
    wspec = lambda shape: pl.BlockSpec(shape, lambda i: tuple(0 for _ in shape))
    return pl.pallas_call(
        functools.partial(_edge_mlp_body, act),
        grid=grid,
        in_specs=[pl.BlockSpec((5, _BE), lambda i: (0, i)),
                  wspec((5, _M)), wspec((_M,)), wspec((_M, _M)),
                  wspec((_M,)), wspec((_M, _H)), wspec((_H,))],
        out_specs=pl.BlockSpec((_BE, _H), lambda i: (i, 0)),
        out_shape=jax.ShapeDtypeStruct((_E, _H), jnp.float32),
    )(in5t, W1, b1, W2, b2, W3, b3)


def _update_body(sp_ref, h_ref, su_ref, num_ref, den_ref,
                 W1a, W1b, W1c, W1d, b1, W2, b2, W3, b3, out_ref):
    f32 = jnp.float32
    su = su_ref[0] + su_ref[1]
    sum_h = num_ref[...] / (den_ref[...] + 1e-16)
    pre = (jnp.dot(sp_ref[...], W1a[...], preferred_element_type=f32)
           + jnp.dot(h_ref[...], W1b[...], preferred_element_type=f32)
           + jnp.dot(su, W1c[...], preferred_element_type=f32)
           + jnp.dot(sum_h, W1d[...], preferred_element_type=f32)
           + b1[...])
    t = jnp.tanh(pre)
    t = jnp.tanh(jnp.dot(t, W2[...], preferred_element_type=f32) + b2[...])
    out_ref[...] = jnp.dot(t, W3[...], preferred_element_type=f32) + b3[...]


def _update_mlp(sp, h, su_part, num, den, p):
    W1 = p['up_W1']
    grid = (_NS // _BN,)
    bspec = pl.BlockSpec((_BN, _H), lambda i: (i, 0))
    wspec = lambda shape: pl.BlockSpec(shape, lambda i: tuple(0 for _ in shape))
    return pl.pallas_call(
        _update_body,
        grid=grid,
        in_specs=[pl.BlockSpec((_BN, 2), lambda i: (i, 0)),
                  bspec,
                  pl.BlockSpec((_NC, _BN, _H), lambda i: (0, i, 0)),
                  bspec, bspec,
                  wspec((2, _M)), wspec((_H, _M)), wspec((_H, _M)), wspec((_H, _M)),
                  wspec((_M,)), wspec((_M, _M)), wspec((_M,)), wspec((_M, _H)), wspec((_H,))],
        out_specs=bspec,
        out_shape=jax.ShapeDtypeStruct((_NS, _H), jnp.float32),
    )(sp, h, su_part, num, den,
      W1[0:2], W1[2:130], W1[130:258], W1[258:386],
      p['up_b1'], p['up_W2'], p['up_b2'], p['up_W3'], p['up_b3'])


_C5 = 2000
_G5 = _C5 // 16
_N5 = _E // _NW // _C5


def _in5_body(stage_ap, ap_hbm, sp_hbm, src_hbm, dst_hbm, dis_hbm, out_hbm,
              ap_v, sp_v, src_v, dst_v, dis_v, out_v):
    cid = lax.axis_index("c")
    sid = lax.axis_index("s")
    wid = sid * _NC + cid
    base = wid * (_E // _NW)

    if stage_ap:
        pltpu.sync_copy(ap_hbm, ap_v)
        src_tab = ap_v
    else:
        src_tab = sp_v
    pltpu.sync_copy(sp_hbm, sp_v)
    dst_tab = sp_v

    iota = lax.iota(jnp.int32, 16)

    def chunk_body(k, _):
        off = base + k * _C5
        pltpu.sync_copy(src_hbm.at[pl.ds(off, _C5)], src_v)
        pltpu.sync_copy(dst_hbm.at[pl.ds(off, _C5)], dst_v)
        pltpu.sync_copy(dis_hbm.at[pl.ds(off, _C5)], dis_v)

        def group_body(g, _):
            isrc = src_v[pl.ds(g * 16, 16)]
            idst = dst_v[pl.ds(g * 16, 16)]
            xs = plsc.load_gather(src_tab, [isrc * 2])
            ys = plsc.load_gather(src_tab, [isrc * 2 + 1])
            xd = plsc.load_gather(dst_tab, [idst * 2])
            yd = plsc.load_gather(dst_tab, [idst * 2 + 1])
            d = dis_v[pl.ds(g * 16, 16)]
            lanes = g * 16 + iota
            plsc.store_scatter(out_v, [lanes], xs)
            plsc.store_scatter(out_v, [_C5 + lanes], ys)
            plsc.store_scatter(out_v, [2 * _C5 + lanes], xd)
            plsc.store_scatter(out_v, [3 * _C5 + lanes], yd)
            plsc.store_scatter(out_v, [4 * _C5 + lanes], d)
            return 0

        lax.fori_loop(0, _G5, group_body, 0)
        for c in range(5):
            pltpu.sync_copy(out_v.at[pl.ds(c * _C5, _C5)],
                            out_hbm.at[pl.ds(c * _E + off, _C5)])
        return 0

    lax.fori_loop(0, _N5, chunk_body, 0)


def _sc_in5(stage_ap, ap_flat, sp_flat, src, dst, dis):
    f = pl.kernel(
        functools.partial(_in5_body, stage_ap),
        out_type=jax.ShapeDtypeStruct((5 * _E,), jnp.float32),
        mesh=_mesh(),
        scratch_types=[
            pltpu.VMEM((2 * _NA,), jnp.float32),
            pltpu.VMEM((2 * _NS,), jnp.float32),
            pltpu.VMEM((_C5,), jnp.int32),
            pltpu.VMEM((_C5,), jnp.int32),
            pltpu.VMEM((_C5,), jnp.float32),
            pltpu.VMEM((5 * _C5,), jnp.float32),
        ],
        compiler_params=pltpu.CompilerParams(needs_layout_passes=False),
    )
    return f(ap_flat, sp_flat, src, dst, dis)


_CA = 80
_BOUNCE = 40


def _zero_acc_stripe(zeros_hbm, acc, sid):
    pltpu.sync_copy(zeros_hbm,
                    acc.at[pl.ds(sid * _ROWS_PER_TILE, _ROWS_PER_TILE)])


def _dump_acc_stripe(acc, sid, dst_hbm_slice_fn):
    off = sid * _ROWS_PER_TILE
    pltpu.sync_copy(acc.at[pl.ds(off, _ROWS_PER_TILE)], dst_hbm_slice_fn(off))


def _pipe_gather_mul_scatter(nch, base, lin_hbm, src_hbm, dst_hbm, tab_hbm, acc,
                             isrc, idst, sidx, rows, lin, si, sd, ss):
    def off(k):
        return base + k * _CA

    pltpu.sync_copy(src_hbm.at[pl.ds(off(0), _CA)], isrc[0])
    pltpu.sync_copy(dst_hbm.at[pl.ds(off(0), _CA)], idst[0])
    pltpu.async_copy(tab_hbm.at[isrc[0]], rows[0], sd[0])
    pltpu.async_copy(lin_hbm.at[pl.ds(off(0), _CA)], lin[0], sd[0])
    pltpu.async_copy(src_hbm.at[pl.ds(off(1), _CA)], isrc[1], si[1])
    pltpu.async_copy(dst_hbm.at[pl.ds(off(1), _CA)], idst[1], si[1])

    def block(k, b):
        o = off(k)
        pltpu.make_async_copy(tab_hbm.at[isrc[b]], rows[b], sd[b]).wait()
        pltpu.make_async_copy(lin_hbm.at[pl.ds(o, _CA)], lin[b], sd[b]).wait()

        nb = 1 - b

        @pl.when(k + 1 < nch)
        def _():
            @pl.when(k >= 1)
            def _():
                pltpu.make_async_copy(rows[nb], acc.at[sidx[nb]], ss[nb]).wait()
            pltpu.make_async_copy(src_hbm.at[pl.ds(off(k + 1), _CA)],
                                  isrc[nb], si[nb]).wait()
            pltpu.make_async_copy(dst_hbm.at[pl.ds(off(k + 1), _CA)],
                                  idst[nb], si[nb]).wait()
            pltpu.async_copy(tab_hbm.at[isrc[nb]], rows[nb], sd[nb])
            pltpu.async_copy(lin_hbm.at[pl.ds(off(k + 1), _CA)], lin[nb], sd[nb])

        @plsc.parallel_loop(0, _CA, unroll=4)
        def _(j):
            for c in range(_H // 16):
                s = pl.ds(c * 16, 16)
                rows[b][j, s] = rows[b][j, s] * lin[b][j, s]

        for c in range(_CA // 16):
            s = pl.ds(c * 16, 16)
            sidx[b][s] = idst[b][s]
        pltpu.async_copy(rows[b], acc.at[sidx[b]], ss[b], add=True)

        @pl.when(k + 2 < nch)
        def _():
            pltpu.async_copy(src_hbm.at[pl.ds(off(k + 2), _CA)], isrc[b], si[b])
            pltpu.async_copy(dst_hbm.at[pl.ds(off(k + 2), _CA)], idst[b], si[b])

    def pair(j, _):
        k0 = 2 * j
        block(k0, 0)

        @pl.when(k0 + 1 < nch)
        def _():
            block(k0 + 1, 1)

        return 0

    lax.fori_loop(0, (nch + 1) // 2, pair, 0)
    pltpu.make_async_copy(rows[0], acc.at[sidx[0]], ss[0]).wait()
    pltpu.make_async_copy(rows[1], acc.at[sidx[1]], ss[1]).wait()


_AGG_SCRATCH = [
    pltpu.VMEM((_CA,), jnp.int32), pltpu.VMEM((_CA,), jnp.int32),
    pltpu.VMEM((_CA,), jnp.int32), pltpu.VMEM((_CA,), jnp.int32),
    pltpu.VMEM((_CA,), jnp.int32), pltpu.VMEM((_CA,), jnp.int32),
    pltpu.VMEM((_CA, _H), jnp.float32), pltpu.VMEM((_CA, _H), jnp.float32),
    pltpu.VMEM((_CA, _H), jnp.float32), pltpu.VMEM((_CA, _H), jnp.float32),
    pltpu.VMEM_SHARED((_NSP, _H), jnp.float32),
    pltpu.SemaphoreType.DMA, pltpu.SemaphoreType.DMA,
    pltpu.SemaphoreType.DMA, pltpu.SemaphoreType.DMA,
    pltpu.SemaphoreType.DMA, pltpu.SemaphoreType.DMA,
]


_EA0 = 51 * _NT * _CA


def _agg_body(gate_hbm, asrc_hbm, adst_hbm, sall_hbm,
              ex_hbm, ssrc_hbm, sdst_hbm, msg_hbm,
              zeros_hbm, su_hbm, num_hbm, den_hbm,
              isrc0, isrc1, idst0, idst1, sidx0, sidx1,
              rows0, rows1, lin0, lin1, acc,
              si0, si1, sd0, sd1, ss0, ss1):
    cid = lax.axis_index("c")
    sid = lax.axis_index("s")
    bufs = ((isrc0, isrc1), (idst0, idst1), (sidx0, sidx1),
            (rows0, rows1), (lin0, lin1),
            (si0, si1), (sd0, sd1), (ss0, ss1))

    _zero_acc_stripe(zeros_hbm, acc, sid)
    plsc.subcore_barrier()

    @pl.when(cid == 0)
    def _():
        _pipe_gather_mul_scatter(_EA0 // _NT // _CA, sid * (_EA0 // _NT),
                                 gate_hbm, asrc_hbm, adst_hbm, sall_hbm, acc,
                                 *bufs)

    @pl.when(cid == 1)
    def _():
        _pipe_gather_mul_scatter((_E - _EA0) // _NT // _CA,
                                 _EA0 + sid * ((_E - _EA0) // _NT),
                                 gate_hbm, asrc_hbm, adst_hbm, sall_hbm, acc,
                                 *bufs)

    plsc.subcore_barrier()
    _dump_acc_stripe(acc, sid,
                     lambda off: su_hbm.at[cid, pl.ds(off, _ROWS_PER_TILE)])

    _zero_acc_stripe(zeros_hbm, acc, sid)
    plsc.subcore_barrier()

    base = sid * (_E // _NT)

    @pl.when(cid == 0)
    def _():
        _pipe_gather_mul_scatter(_E // _NT // _CA, base,
                                 ex_hbm, ssrc_hbm, sdst_hbm, msg_hbm, acc,
                                 *bufs)

    @pl.when(cid == 1)
    def _():
        nch = _E // _NT // _CA
        idx3 = (idst0, idst1, isrc0)
        lin3 = (lin0, lin1, rows0)
        ld3 = (sd0, sd1, si0)
        sc3 = (ss0, ss1, si1)

        def off(k):
            return base + k * _CA

        pltpu.sync_copy(sdst_hbm.at[pl.ds(off(0), _CA)], idx3[0])
        pltpu.sync_copy(ex_hbm.at[pl.ds(off(0), _CA)], lin3[0].at[pl.ds(0, _CA)])
        pltpu.async_copy(sdst_hbm.at[pl.ds(off(1), _CA)], idx3[1], ld3[1])
        pltpu.async_copy(ex_hbm.at[pl.ds(off(1), _CA)], lin3[1].at[pl.ds(0, _CA)], ld3[1])

        def den_block(k, r):
            @pl.when(k >= 1)
            def _():
                pltpu.make_async_copy(sdst_hbm.at[pl.ds(off(k), _CA)],
                                      idx3[r], ld3[r]).wait()
                pltpu.make_async_copy(ex_hbm.at[pl.ds(off(k), _CA)],
                                      lin3[r].at[pl.ds(0, _CA)], ld3[r]).wait()
            pltpu.async_copy(lin3[r].at[pl.ds(0, _CA)], acc.at[idx3[r]],
                             sc3[r], add=True)
            r2 = (r + 2) % 3

            @pl.when(k + 2 < nch)
            def _():
                @pl.when(k >= 1)
                def _():
                    pltpu.make_async_copy(lin3[r2].at[pl.ds(0, _CA)],
                                          acc.at[idx3[r2]], sc3[r2]).wait()
                pltpu.async_copy(sdst_hbm.at[pl.ds(off(k + 2), _CA)],
                                 idx3[r2], ld3[r2])
                pltpu.async_copy(ex_hbm.at[pl.ds(off(k + 2), _CA)],
                                 lin3[r2].at[pl.ds(0, _CA)], ld3[r2])

        def triple(j, _):
            k0 = 3 * j
            for r in range(3):
                @pl.when(k0 + r < nch)
                def _():
                    den_block(k0 + r, r)
            return 0

        lax.fori_loop(0, (nch + 2) // 3, triple, 0)
        for r in range(3):
            pltpu.make_async_copy(lin3[r].at[pl.ds(0, _CA)],
                                  acc.at[idx3[r]], sc3[r]).wait()

    plsc.subcore_barrier()

    @pl.when(cid == 0)
    def _():
        _dump_acc_stripe(acc, sid,
                         lambda off: num_hbm.at[pl.ds(off, _ROWS_PER_TILE)])

    @pl.when(cid == 1)
    def _():
        _dump_acc_stripe(acc, sid,
                         lambda off: den_hbm.at[pl.ds(off, _ROWS_PER_TILE)])


def _sc_agg(gate, a_src, a_dst, s_all, ex, s_src, s_dst, msg_all, zeros_stripe):
    f = pl.kernel(
        _agg_body,
        out_type=[jax.ShapeDtypeStruct((_NC, _NSP, _H), jnp.float32),
                  jax.ShapeDtypeStruct((_NSP, _H), jnp.float32),
                  jax.ShapeDtypeStruct((_NSP, _H), jnp.float32)],
        mesh=_mesh(),
        scratch_types=list(_AGG_SCRATCH),
        compiler_params=pltpu.CompilerParams(needs_layout_passes=False),
    )
    return f(gate, a_src, a_dst, s_all, ex, s_src, s_dst, msg_all, zeros_stripe)


def kernel(h, u, state_pos, action_pos, a2s_src, a2s_dst, a2s_dis,
           s2s_src, s2s_dst, s2s_dis, params):
    p = params
    ap_flat = jnp.reshape(action_pos, (-1,))
    sp_flat = jnp.reshape(state_pos, (-1,))
    a_dis = jnp.reshape(a2s_dis, (-1,))
    s_dis = jnp.reshape(s2s_dis, (-1,))

    in5a_flat = _sc_in5(True, ap_flat, sp_flat, a2s_src, a2s_dst, a_dis)
    in5a = jnp.reshape(in5a_flat, (5, _E))
    gate = _edge_mlp(in5a, p['ud_W1'], p['ud_b1'], p['ud_W2'], p['ud_b2'],
                     p['ud_W3'], p['ud_b3'], 'sigmoid')

    in5s_flat = _sc_in5(False, ap_flat, sp_flat, s2s_src, s2s_dst, s_dis)
    in5s = jnp.reshape(in5s_flat, (5, _E))
    ex = _edge_mlp(in5s, p['hd_W1'], p['hd_b1'], p['hd_W2'], p['hd_b2'],
                   p['hd_W3'], p['hd_b3'], 'exp')

    s_all, msg_all = _node_mlps(u, h, p)

    zeros_stripe = jnp.zeros((_ROWS_PER_TILE, _H), jnp.float32)
    su_part, num, den = _sc_agg(gate, a2s_src, a2s_dst, s_all,
                                ex, s2s_src, s2s_dst, msg_all, zeros_stripe)

    return _update_mlp(state_pos, h, su_part, num, den, p)


# speedup vs baseline: 1.3741x; 1.0056x over previous
import functools
import jax
import jax.numpy as jnp
from jax import lax
from jax.experimental import pallas as pl
from jax.experimental.pallas import tpu as pltpu
from jax.experimental.pallas import tpu_sc as plsc

_NS = 10000
_NA = 10000
_E = 320000
_H = 128
_M = 64

_BN = 2000
_BE = 12800

_NC = 2
_NT = 16
_NW = _NC * _NT
_NSP = 10240
_ROWS_PER_TILE = _NSP // _NT

_mesh = lambda: plsc.VectorSubcoreMesh(core_axis_name="c", subcore_axis_name="s")


def _node_mlp_body(u_ref, h_ref,
                   uuW1, uuW2, uuW3,
                   hhW1, hhb1, hhW2, hhb2, hhW3, hhb3,
                   s_out, msg_out):
    f32 = jnp.float32
    s = jnp.tanh(jnp.dot(u_ref[...], uuW1[...], preferred_element_type=f32))
    s = jnp.tanh(jnp.dot(s, uuW2[...], preferred_element_type=f32))
    s_out[...] = jnp.dot(s, uuW3[...], preferred_element_type=f32)
    m = jnp.tanh(jnp.dot(h_ref[...], hhW1[...], preferred_element_type=f32) + hhb1[...])
    m = jnp.tanh(jnp.dot(m, hhW2[...], preferred_element_type=f32) + hhb2[...])
    msg_out[...] = jnp.dot(m, hhW3[...], preferred_element_type=f32) + hhb3[...]


def _node_mlps(u, h, p):
    grid = (_NS // _BN,)
    bspec = pl.BlockSpec((_BN, _H), lambda i: (i, 0))
    wspec = lambda shape: pl.BlockSpec(shape, lambda i: tuple(0 for _ in shape))
    out_shape = [jax.ShapeDtypeStruct((_NS, _H), jnp.float32)] * 2
    return pl.pallas_call(
        _node_mlp_body,
        grid=grid,
        in_specs=[bspec, bspec,
                  wspec((_H, _M)), wspec((_M, _M)), wspec((_M, _H)),
                  wspec((_H, _M)), wspec((_M,)), wspec((_M, _M)),
                  wspec((_M,)), wspec((_M, _H)), wspec((_H,))],
        out_specs=[bspec, bspec],
        out_shape=out_shape,
    )(u, h, p['uu_W1'], p['uu_W2'], p['uu_W3'],
      p['hh_W1'], p['hh_b1'], p['hh_W2'], p['hh_b2'], p['hh_W3'], p['hh_b3'])


def _edge_mlp_body(act, in5_ref, W1, b1, W2, b2, W3, b3, out_ref):
    f32 = jnp.float32
    pre = lax.dot_general(in5_ref[...], W1[...],
                          dimension_numbers=(((0,), (0,)), ((), ())),
                          preferred_element_type=f32)
    t = jnp.tanh(pre + b1[...])
    t = jnp.tanh(jnp.dot(t, W2[...], preferred_element_type=f32) + b2[...])
    z = jnp.dot(t, W3[...], preferred_element_type=f32) + b3[...]
    if act == 'sigmoid':
        out_ref[...] = jax.nn.sigmoid(z)
    else:
        out_ref[...] = jnp.exp(z)


def _edge_mlp(in5t, W1, b1, W2, b2, W3, b3, act):
    grid = (_E // _BE,)
    wspec = lambda shape: pl.BlockSpec(shape, lambda i: tuple(0 for _ in shape))
    return pl.pallas_call(
        functools.partial(_edge_mlp_body, act),
        grid=grid,
        in_specs=[pl.BlockSpec((5, _BE), lambda i: (0, i)),
                  wspec((5, _M)), wspec((_M,)), wspec((_M, _M)),
                  wspec((_M,)), wspec((_M, _H)), wspec((_H,))],
        out_specs=pl.BlockSpec((_BE, _H), lambda i: (i, 0)),
        out_shape=jax.ShapeDtypeStruct((_E, _H), jnp.float32),
    )(in5t, W1, b1, W2, b2, W3, b3)


def _update_body(sp_ref, h_ref, su_ref, num_ref, den_ref,
                 W1a, W1b, W1c, W1d, b1, W2, b2, W3, b3, out_ref):
    f32 = jnp.float32
    su = su_ref[0] + su_ref[1]
    sum_h = num_ref[...] / (den_ref[...] + 1e-16)
    pre = (jnp.dot(sp_ref[...], W1a[...], preferred_element_type=f32)
           + jnp.dot(h_ref[...], W1b[...], preferred_element_type=f32)
           + jnp.dot(su, W1c[...], preferred_element_type=f32)
           + jnp.dot(sum_h, W1d[...], preferred_element_type=f32)
           + b1[...])
    t = jnp.tanh(pre)
    t = jnp.tanh(jnp.dot(t, W2[...], preferred_element_type=f32) + b2[...])
    out_ref[...] = jnp.dot(t, W3[...], preferred_element_type=f32) + b3[...]


def _update_mlp(sp, h, su_part, num, den, p):
    W1 = p['up_W1']
    grid = (_NS // _BN,)
    bspec = pl.BlockSpec((_BN, _H), lambda i: (i, 0))
    wspec = lambda shape: pl.BlockSpec(shape, lambda i: tuple(0 for _ in shape))
    return pl.pallas_call(
        _update_body,
        grid=grid,
        in_specs=[pl.BlockSpec((_BN, 2), lambda i: (i, 0)),
                  bspec,
                  pl.BlockSpec((_NC, _BN, _H), lambda i: (0, i, 0)),
                  bspec, bspec,
                  wspec((2, _M)), wspec((_H, _M)), wspec((_H, _M)), wspec((_H, _M)),
                  wspec((_M,)), wspec((_M, _M)), wspec((_M,)), wspec((_M, _H)), wspec((_H,))],
        out_specs=bspec,
        out_shape=jax.ShapeDtypeStruct((_NS, _H), jnp.float32),
    )(sp, h, su_part, num, den,
      W1[0:2], W1[2:130], W1[130:258], W1[258:386],
      p['up_b1'], p['up_W2'], p['up_b2'], p['up_W3'], p['up_b3'])


_C5 = 2000
_G5 = _C5 // 16
_N5 = _E // _NW // _C5


def _in5_body(stage_ap, ap_hbm, sp_hbm, src_hbm, dst_hbm, dis_hbm, out_hbm,
              ap_v, sp_v, src_v, dst_v, dis_v, out_v):
    cid = lax.axis_index("c")
    sid = lax.axis_index("s")
    wid = sid * _NC + cid
    base = wid * (_E // _NW)

    if stage_ap:
        pltpu.sync_copy(ap_hbm, ap_v)
        src_tab = ap_v
    else:
        src_tab = sp_v
    pltpu.sync_copy(sp_hbm, sp_v)
    dst_tab = sp_v

    iota = lax.iota(jnp.int32, 16)

    def chunk_body(k, _):
        off = base + k * _C5
        pltpu.sync_copy(src_hbm.at[pl.ds(off, _C5)], src_v)
        pltpu.sync_copy(dst_hbm.at[pl.ds(off, _C5)], dst_v)
        pltpu.sync_copy(dis_hbm.at[pl.ds(off, _C5)], dis_v)

        def group_body(g, _):
            isrc = src_v[pl.ds(g * 16, 16)]
            idst = dst_v[pl.ds(g * 16, 16)]
            xs = plsc.load_gather(src_tab, [isrc * 2])
            ys = plsc.load_gather(src_tab, [isrc * 2 + 1])
            xd = plsc.load_gather(dst_tab, [idst * 2])
            yd = plsc.load_gather(dst_tab, [idst * 2 + 1])
            d = dis_v[pl.ds(g * 16, 16)]
            lanes = g * 16 + iota
            plsc.store_scatter(out_v, [lanes], xs)
            plsc.store_scatter(out_v, [_C5 + lanes], ys)
            plsc.store_scatter(out_v, [2 * _C5 + lanes], xd)
            plsc.store_scatter(out_v, [3 * _C5 + lanes], yd)
            plsc.store_scatter(out_v, [4 * _C5 + lanes], d)
            return 0

        lax.fori_loop(0, _G5, group_body, 0)
        for c in range(5):
            pltpu.sync_copy(out_v.at[pl.ds(c * _C5, _C5)],
                            out_hbm.at[pl.ds(c * _E + off, _C5)])
        return 0

    lax.fori_loop(0, _N5, chunk_body, 0)


def _sc_in5(stage_ap, ap_flat, sp_flat, src, dst, dis):
    f = pl.kernel(
        functools.partial(_in5_body, stage_ap),
        out_type=jax.ShapeDtypeStruct((5 * _E,), jnp.float32),
        mesh=_mesh(),
        scratch_types=[
            pltpu.VMEM((2 * _NA,), jnp.float32),
            pltpu.VMEM((2 * _NS,), jnp.float32),
            pltpu.VMEM((_C5,), jnp.int32),
            pltpu.VMEM((_C5,), jnp.int32),
            pltpu.VMEM((_C5,), jnp.float32),
            pltpu.VMEM((5 * _C5,), jnp.float32),
        ],
        compiler_params=pltpu.CompilerParams(needs_layout_passes=False),
    )
    return f(ap_flat, sp_flat, src, dst, dis)


_CA = 80


def _zero_acc_stripe(zeros_hbm, acc, sid):
    pltpu.sync_copy(zeros_hbm,
                    acc.at[pl.ds(sid * _ROWS_PER_TILE, _ROWS_PER_TILE)])


def _dump_acc_stripe(acc, sid, dst_hbm_slice_fn):
    off = sid * _ROWS_PER_TILE
    pltpu.sync_copy(acc.at[pl.ds(off, _ROWS_PER_TILE)], dst_hbm_slice_fn(off))


def _pipe_gather_mul_scatter(nch, base, lin_hbm, src_hbm, dst_hbm, tab_hbm, acc,
                             isrc, idst, sidx, rows, lin, si, sd, ss):
    def off(k):
        return base + k * _CA

    pltpu.sync_copy(src_hbm.at[pl.ds(off(0), _CA)], isrc[0])
    pltpu.sync_copy(dst_hbm.at[pl.ds(off(0), _CA)], idst[0])
    pltpu.async_copy(tab_hbm.at[isrc[0]], rows[0], sd[0])
    pltpu.async_copy(lin_hbm.at[pl.ds(off(0), _CA)], lin[0], sd[0])
    pltpu.async_copy(src_hbm.at[pl.ds(off(1), _CA)], isrc[1], si[1])
    pltpu.async_copy(dst_hbm.at[pl.ds(off(1), _CA)], idst[1], si[1])

    def block(k, b):
        o = off(k)
        pltpu.make_async_copy(tab_hbm.at[isrc[b]], rows[b], sd[b]).wait()
        pltpu.make_async_copy(lin_hbm.at[pl.ds(o, _CA)], lin[b], sd[b]).wait()

        nb = 1 - b

        @pl.when(k + 1 < nch)
        def _():
            @pl.when(k >= 1)
            def _():
                pltpu.make_async_copy(rows[nb], acc.at[sidx[nb]], ss[nb]).wait()
            pltpu.make_async_copy(src_hbm.at[pl.ds(off(k + 1), _CA)],
                                  isrc[nb], si[nb]).wait()
            pltpu.make_async_copy(dst_hbm.at[pl.ds(off(k + 1), _CA)],
                                  idst[nb], si[nb]).wait()
            pltpu.async_copy(tab_hbm.at[isrc[nb]], rows[nb], sd[nb])
            pltpu.async_copy(lin_hbm.at[pl.ds(off(k + 1), _CA)], lin[nb], sd[nb])

        @plsc.parallel_loop(0, _CA, unroll=4)
        def _(j):
            for c in range(_H // 16):
                s = pl.ds(c * 16, 16)
                rows[b][j, s] = rows[b][j, s] * lin[b][j, s]

        for c in range(_CA // 16):
            s = pl.ds(c * 16, 16)
            sidx[b][s] = idst[b][s]
        pltpu.async_copy(rows[b], acc.at[sidx[b]], ss[b], add=True)

        @pl.when(k + 2 < nch)
        def _():
            pltpu.async_copy(src_hbm.at[pl.ds(off(k + 2), _CA)], isrc[b], si[b])
            pltpu.async_copy(dst_hbm.at[pl.ds(off(k + 2), _CA)], idst[b], si[b])

    def pair(j, _):
        k0 = 2 * j
        block(k0, 0)

        @pl.when(k0 + 1 < nch)
        def _():
            block(k0 + 1, 1)

        return 0

    lax.fori_loop(0, (nch + 1) // 2, pair, 0)
    pltpu.make_async_copy(rows[0], acc.at[sidx[0]], ss[0]).wait()
    pltpu.make_async_copy(rows[1], acc.at[sidx[1]], ss[1]).wait()


_AGG_SCRATCH = [
    pltpu.VMEM((_CA,), jnp.int32), pltpu.VMEM((_CA,), jnp.int32),
    pltpu.VMEM((_CA,), jnp.int32), pltpu.VMEM((_CA,), jnp.int32),
    pltpu.VMEM((_CA,), jnp.int32), pltpu.VMEM((_CA,), jnp.int32),
    pltpu.VMEM((_CA, _H), jnp.float32), pltpu.VMEM((_CA, _H), jnp.float32),
    pltpu.VMEM((_CA, _H), jnp.float32), pltpu.VMEM((_CA, _H), jnp.float32),
    pltpu.VMEM_SHARED((_NSP, _H), jnp.float32),
    pltpu.SemaphoreType.DMA, pltpu.SemaphoreType.DMA,
    pltpu.SemaphoreType.DMA, pltpu.SemaphoreType.DMA,
    pltpu.SemaphoreType.DMA, pltpu.SemaphoreType.DMA,
]


_EA0 = 51 * _NT * _CA


def _agg_body(gate_hbm, asrc_hbm, adst_hbm, sall_hbm,
              ex_hbm, ssrc_hbm, sdst_hbm, msg_hbm,
              zeros_hbm, su_hbm, num_hbm, den_hbm,
              isrc0, isrc1, idst0, idst1, sidx0, sidx1,
              rows0, rows1, lin0, lin1, acc,
              si0, si1, sd0, sd1, ss0, ss1):
    cid = lax.axis_index("c")
    sid = lax.axis_index("s")
    bufs = ((isrc0, isrc1), (idst0, idst1), (sidx0, sidx1),
            (rows0, rows1), (lin0, lin1),
            (si0, si1), (sd0, sd1), (ss0, ss1))

    _zero_acc_stripe(zeros_hbm, acc, sid)
    plsc.subcore_barrier()

    @pl.when(cid == 0)
    def _():
        _pipe_gather_mul_scatter(_EA0 // _NT // _CA, sid * (_EA0 // _NT),
                                 gate_hbm, asrc_hbm, adst_hbm, sall_hbm, acc,
                                 *bufs)

    @pl.when(cid == 1)
    def _():
        _pipe_gather_mul_scatter((_E - _EA0) // _NT // _CA,
                                 _EA0 + sid * ((_E - _EA0) // _NT),
                                 gate_hbm, asrc_hbm, adst_hbm, sall_hbm, acc,
                                 *bufs)

    plsc.subcore_barrier()
    _dump_acc_stripe(acc, sid,
                     lambda off: su_hbm.at[cid, pl.ds(off, _ROWS_PER_TILE)])

    _zero_acc_stripe(zeros_hbm, acc, sid)
    plsc.subcore_barrier()

    base = sid * (_E // _NT)

    @pl.when(cid == 0)
    def _():
        _pipe_gather_mul_scatter(_E // _NT // _CA, base,
                                 ex_hbm, ssrc_hbm, sdst_hbm, msg_hbm, acc,
                                 *bufs)

    @pl.when(cid == 1)
    def _():
        nch = _E // _NT // _CA
        idx3 = (idst0, idst1, isrc0)
        lin3 = (lin0, lin1, rows0)
        ld3 = (sd0, sd1, si0)
        sc3 = (ss0, ss1, si1)

        def off(k):
            return base + k * _CA

        pltpu.sync_copy(sdst_hbm.at[pl.ds(off(0), _CA)], idx3[0])
        pltpu.sync_copy(ex_hbm.at[pl.ds(off(0), _CA)], lin3[0].at[pl.ds(0, _CA)])
        pltpu.async_copy(sdst_hbm.at[pl.ds(off(1), _CA)], idx3[1], ld3[1])
        pltpu.async_copy(ex_hbm.at[pl.ds(off(1), _CA)], lin3[1].at[pl.ds(0, _CA)], ld3[1])

        def den_block(k, r):
            @pl.when(k >= 1)
            def _():
                pltpu.make_async_copy(sdst_hbm.at[pl.ds(off(k), _CA)],
                                      idx3[r], ld3[r]).wait()
                pltpu.make_async_copy(ex_hbm.at[pl.ds(off(k), _CA)],
                                      lin3[r].at[pl.ds(0, _CA)], ld3[r]).wait()
            pltpu.async_copy(lin3[r].at[pl.ds(0, _CA)], acc.at[idx3[r]],
                             sc3[r], add=True)
            r2 = (r + 2) % 3

            @pl.when(k + 2 < nch)
            def _():
                @pl.when(k >= 1)
                def _():
                    pltpu.make_async_copy(lin3[r2].at[pl.ds(0, _CA)],
                                          acc.at[idx3[r2]], sc3[r2]).wait()
                pltpu.async_copy(sdst_hbm.at[pl.ds(off(k + 2), _CA)],
                                 idx3[r2], ld3[r2])
                pltpu.async_copy(ex_hbm.at[pl.ds(off(k + 2), _CA)],
                                 lin3[r2].at[pl.ds(0, _CA)], ld3[r2])

        def triple(j, _):
            k0 = 3 * j
            for r in range(3):
                @pl.when(k0 + r < nch)
                def _():
                    den_block(k0 + r, r)
            return 0

        lax.fori_loop(0, (nch + 2) // 3, triple, 0)
        for r in range(3):
            pltpu.make_async_copy(lin3[r].at[pl.ds(0, _CA)],
                                  acc.at[idx3[r]], sc3[r]).wait()

    plsc.subcore_barrier()

    @pl.when(cid == 0)
    def _():
        _dump_acc_stripe(acc, sid,
                         lambda off: num_hbm.at[pl.ds(off, _ROWS_PER_TILE)])

    @pl.when(cid == 1)
    def _():
        _dump_acc_stripe(acc, sid,
                         lambda off: den_hbm.at[pl.ds(off, _ROWS_PER_TILE)])


def _sc_agg(gate, a_src, a_dst, s_all, ex, s_src, s_dst, msg_all, zeros_stripe):
    f = pl.kernel(
        _agg_body,
        out_type=[jax.ShapeDtypeStruct((_NC, _NSP, _H), jnp.float32),
                  jax.ShapeDtypeStruct((_NSP, _H), jnp.float32),
                  jax.ShapeDtypeStruct((_NSP, _H), jnp.float32)],
        mesh=_mesh(),
        scratch_types=list(_AGG_SCRATCH),
        compiler_params=pltpu.CompilerParams(needs_layout_passes=False),
    )
    return f(gate, a_src, a_dst, s_all, ex, s_src, s_dst, msg_all, zeros_stripe)


def kernel(h, u, state_pos, action_pos, a2s_src, a2s_dst, a2s_dis,
           s2s_src, s2s_dst, s2s_dis, params):
    p = params
    ap_flat = jnp.reshape(action_pos, (-1,))
    sp_flat = jnp.reshape(state_pos, (-1,))
    a_dis = jnp.reshape(a2s_dis, (-1,))
    s_dis = jnp.reshape(s2s_dis, (-1,))

    in5a_flat = _sc_in5(True, ap_flat, sp_flat, a2s_src, a2s_dst, a_dis)
    in5a = jnp.reshape(in5a_flat, (5, _E))
    gate = _edge_mlp(in5a, p['ud_W1'], p['ud_b1'], p['ud_W2'], p['ud_b2'],
                     p['ud_W3'], p['ud_b3'], 'sigmoid')

    in5s_flat = _sc_in5(False, ap_flat, sp_flat, s2s_src, s2s_dst, s_dis)
    in5s = jnp.reshape(in5s_flat, (5, _E))
    ex = _edge_mlp(in5s, p['hd_W1'], p['hd_b1'], p['hd_W2'], p['hd_b2'],
                   p['hd_W3'], p['hd_b3'], 'exp')

    s_all, msg_all = _node_mlps(u, h, p)

    zeros_stripe = jnp.zeros((_ROWS_PER_TILE, _H), jnp.float32)
    su_part, num, den = _sc_agg(gate, a2s_src, a2s_dst, s_all,
                                ex, s2s_src, s2s_dst, msg_all, zeros_stripe)

    return _update_mlp(state_pos, h, su_part, num, den, p)
